# baseline (speedup 1.0000x reference)
import numpy as np
import jax
import jax.numpy as jnp
from jax.experimental import pallas as pl
from jax.experimental.pallas import tpu as pltpu

EPS = 1e-5


def _stream_kernel(x_ref, w_ref, idx_ref, hist_ref):
    x = x_ref[...]
    hist_ref[...] = jnp.sum(x, axis=0, keepdims=True)[None]

    d = jnp.dot(x.astype(jnp.bfloat16), w_ref[...],
                preferred_element_type=jnp.float32)
    idx_ref[...] = jnp.sum(d, axis=1, keepdims=True)


def _lookup_kernel(idx_ref, hist_ref, tw_ref, w1_ref, b1_ref, g1_ref,
                   be1_ref, w2_ref, b2_ref, g2_ref, be2_ref, a1_ref, a2_ref,
                   o_ref, table_ref):
    rows = o_ref.shape[0]
    classes = w1_ref.shape[0]
    t = pl.program_id(1)

    @pl.when(t == 0)
    def _build_table():
        n_rows = jnp.sum(hist_ref[...])
        inv_n = 1.0 / n_rows

        cnt_row = jnp.sum(hist_ref[:, 0, :], axis=0, keepdims=True)
        hi = jnp.floor(cnt_row * (1.0 / 64.0))
        lo = cnt_row - 64.0 * hi
        stacked = jnp.concatenate([hi, lo], axis=0)
        cnt_full = jax.lax.dot_general(
            stacked.astype(jnp.bfloat16), tw_ref[...],
            (((0,), (0,)), ((), ())),
            preferred_element_type=jnp.float32)
        cnt = cnt_full[:, 0:1]

        a1 = a1_ref[0, 0]
        a2 = a2_ref[0, 0]

        h = w1_ref[...] + b1_ref[...]
        mean1 = jnp.sum(h * cnt, axis=0, keepdims=True) * inv_n
        d = h - mean1
        var1 = jnp.sum(d * d * cnt, axis=0, keepdims=True) * inv_n
        scale1 = jax.lax.rsqrt(var1 + EPS) * g1_ref[...]
        z = d * scale1 + be1_ref[...]
        z = jnp.where(z > 0, z, a1 * z)

        y = jnp.dot(z, w2_ref[...],
                    preferred_element_type=jnp.float32) + b2_ref[...]
        mean2 = jnp.sum(y * cnt, axis=0, keepdims=True) * inv_n
        e = y - mean2
        var2 = jnp.sum(e * e * cnt, axis=0, keepdims=True) * inv_n
        scale2 = jax.lax.rsqrt(var2 + EPS) * g2_ref[...]
        u = e * scale2 + be2_ref[...]
        table_ref[...] = jnp.where(u > 0, u, a2 * u).astype(jnp.bfloat16)

    iv = idx_ref[...].astype(jnp.int32)
    lane = jax.lax.broadcasted_iota(jnp.int32, (rows, classes), 1)
    onehot = (lane == iv).astype(jnp.bfloat16)
    o_ref[...] = jnp.dot(onehot, table_ref[...],
                         preferred_element_type=jnp.float32)


def kernel(pos_onehot, w1, b1, g1, be1, a1, w2, b2, g2, be2, a2):
    b, l, classes = pos_onehot.shape
    out_dim = w2.shape[1]
    n = b * l

    x = pos_onehot.reshape(n, classes)

    tiles = 8
    rows = n // tiles
    t_inner = tiles // 2

    cgrid = np.arange(classes)
    wnp = np.zeros((classes, 128), np.float32)
    wnp[:, 0] = (cgrid // 128) * 128
    wnp[:, 1] = cgrid % 128
    w_idx = jnp.asarray(wnp, dtype=jnp.bfloat16)

    const = lambda shape: pl.BlockSpec(shape, lambda i, s=len(shape): (0,) * s)

    idx, hist = pl.pallas_call(
        _stream_kernel,
        out_shape=(jax.ShapeDtypeStruct((n, 1), jnp.float32),
                   jax.ShapeDtypeStruct((tiles, 1, classes), jnp.float32)),
        grid=(tiles,),
        in_specs=[
            pl.BlockSpec((rows, classes), lambda i: (i, 0)),
            const((classes, 128)),
        ],
        out_specs=(pl.BlockSpec((rows, 1), lambda i: (i, 0)),
                   pl.BlockSpec((1, 1, classes), lambda i: (i, 0, 0))),
        compiler_params=pltpu.CompilerParams(
            dimension_semantics=("parallel",)),
    )(x, w_idx)

    twnp = np.zeros((2, 128), np.float32)
    twnp[0, :] = 64.0
    twnp[1, :] = 1.0
    t_w = jnp.asarray(twnp, dtype=jnp.bfloat16)

    const2 = lambda shape: pl.BlockSpec(
        shape, lambda c, t, s=len(shape): (0,) * s)
    smem = pl.BlockSpec(memory_space=pltpu.MemorySpace.SMEM)

    out = pl.pallas_call(
        _lookup_kernel,
        out_shape=jax.ShapeDtypeStruct((n, out_dim), jnp.float32),
        grid=(2, t_inner),
        in_specs=[
            pl.BlockSpec((rows, 1), lambda c, t: (c * t_inner + t, 0)),
            const2(hist.shape), const2(t_w.shape), const2(w1.shape),
            const2(b1.shape), const2(g1.shape), const2(be1.shape),
            const2(w2.shape), const2(b2.shape), const2(g2.shape),
            const2(be2.shape), smem, smem,
        ],
        out_specs=pl.BlockSpec((rows, out_dim),
                               lambda c, t: (c * t_inner + t, 0)),
        scratch_shapes=[pltpu.VMEM((classes, out_dim), jnp.bfloat16)],
        compiler_params=pltpu.CompilerParams(
            dimension_semantics=("parallel", "arbitrary")),
    )(idx, hist, t_w, w1, b1, g1, be1, w2, b2, g2, be2, a1, a2)

    return out.reshape(b, l, out_dim)


# speedup vs baseline: 1.1481x; 1.1481x over previous
import numpy as np
import jax
import jax.numpy as jnp
from jax.experimental import pallas as pl
from jax.experimental.pallas import tpu as pltpu

EPS = 1e-5


def _fused_kernel(tiles, x_ref, w_ref, tw_ref, w1_ref, b1_ref, g1_ref,
                  be1_ref, w2_ref, b2_ref, g2_ref, be2_ref, a1_ref, a2_ref,
                  o_ref, idx_scr, hist_scr, table_scr):
    t = pl.program_id(0)
    rows = o_ref.shape[1]
    classes = w1_ref.shape[0]

    @pl.when(t == 0)
    def _init():
        hist_scr[...] = jnp.zeros_like(hist_scr)

    @pl.when(t < tiles)
    def _stream():
        x = x_ref[0]
        hist_scr[...] += jnp.sum(x, axis=0, keepdims=True)
        d = jnp.dot(x.astype(jnp.bfloat16), w_ref[...],
                    preferred_element_type=jnp.float32)
        off = pl.multiple_of(t * rows, rows)
        idx_scr[pl.ds(off, rows), :] = jnp.sum(d, axis=1, keepdims=True)

    @pl.when(t == tiles)
    def _build_table():
        inv_n = 1.0 / jnp.sum(hist_scr[...])
        cnt_row = hist_scr[...]
        hi = jnp.floor(cnt_row * (1.0 / 64.0))
        lo = cnt_row - 64.0 * hi
        stacked = jnp.concatenate([hi, lo], axis=0)
        cnt_full = jax.lax.dot_general(
            stacked.astype(jnp.bfloat16), tw_ref[...],
            (((0,), (0,)), ((), ())),
            preferred_element_type=jnp.float32)
        cnt = cnt_full[:, 0:1]

        a1 = a1_ref[0, 0]
        a2 = a2_ref[0, 0]

        h = w1_ref[...] + b1_ref[...]
        mean1 = jnp.sum(h * cnt, axis=0, keepdims=True) * inv_n
        d = h - mean1
        var1 = jnp.sum(d * d * cnt, axis=0, keepdims=True) * inv_n
        scale1 = jax.lax.rsqrt(var1 + EPS) * g1_ref[...]
        z = d * scale1 + be1_ref[...]
        z = jnp.where(z > 0, z, a1 * z)

        y = jnp.dot(z, w2_ref[...],
                    preferred_element_type=jnp.float32) + b2_ref[...]
        mean2 = jnp.sum(y * cnt, axis=0, keepdims=True) * inv_n
        e = y - mean2
        var2 = jnp.sum(e * e * cnt, axis=0, keepdims=True) * inv_n
        scale2 = jax.lax.rsqrt(var2 + EPS) * g2_ref[...]
        u = e * scale2 + be2_ref[...]
        table_scr[...] = jnp.where(u > 0, u, a2 * u).astype(jnp.bfloat16)

    @pl.when(t >= tiles)
    def _gather():
        off = pl.multiple_of((t - tiles) * rows, rows)
        iv = idx_scr[pl.ds(off, rows), :].astype(jnp.int32)
        lane = jax.lax.broadcasted_iota(jnp.int32, (rows, classes), 1)
        onehot = (lane == iv).astype(jnp.bfloat16)
        o_ref[0] = jnp.dot(onehot, table_scr[...],
                           preferred_element_type=jnp.float32)


def kernel(pos_onehot, w1, b1, g1, be1, a1, w2, b2, g2, be2, a2):
    b, l, classes = pos_onehot.shape
    out_dim = w2.shape[1]
    n = b * l

    tiles = b
    rows = l

    cgrid = np.arange(classes)
    wnp = np.zeros((classes, 128), np.float32)
    wnp[:, 0] = (cgrid // 128) * 128
    wnp[:, 1] = cgrid % 128
    w_idx = jnp.asarray(wnp, dtype=jnp.bfloat16)

    twnp = np.zeros((2, 128), np.float32)
    twnp[0, :] = 64.0
    twnp[1, :] = 1.0
    t_w = jnp.asarray(twnp, dtype=jnp.bfloat16)

    const = lambda shape: pl.BlockSpec(shape, lambda i, s=len(shape): (0,) * s)
    smem = pl.BlockSpec(memory_space=pltpu.MemorySpace.SMEM)

    import functools
    body = functools.partial(_fused_kernel, tiles)

    out = pl.pallas_call(
        body,
        out_shape=jax.ShapeDtypeStruct((b, l, out_dim), jnp.float32),
        grid=(2 * tiles,),
        in_specs=[
            pl.BlockSpec((1, l, classes),
                         lambda t: (jnp.minimum(t, tiles - 1), 0, 0)),
            const(w_idx.shape), const(t_w.shape), const(w1.shape),
            const(b1.shape), const(g1.shape), const(be1.shape),
            const(w2.shape), const(b2.shape), const(g2.shape),
            const(be2.shape), smem, smem,
        ],
        out_specs=pl.BlockSpec((1, l, out_dim),
                               lambda t: (jnp.maximum(t - tiles, 0), 0, 0)),
        scratch_shapes=[pltpu.VMEM((n, 1), jnp.float32),
                        pltpu.VMEM((1, classes), jnp.float32),
                        pltpu.VMEM((classes, out_dim), jnp.bfloat16)],
        compiler_params=pltpu.CompilerParams(
            dimension_semantics=("arbitrary",)),
    )(pos_onehot, w_idx, t_w, w1, b1, g1, be1, w2, b2, g2, be2, a1, a2)

    return out


# speedup vs baseline: 1.1511x; 1.0026x over previous
import numpy as np
import jax
import jax.numpy as jnp
from jax.experimental import pallas as pl
from jax.experimental.pallas import tpu as pltpu

EPS = 1e-5


def _fused_kernel(tiles, x_ref, w1_ref, b1_ref, g1_ref,
                  be1_ref, w2_ref, b2_ref, g2_ref, be2_ref, a1_ref, a2_ref,
                  o_ref, idx_scr, hist_scr, table_scr, w_scr):
    t = pl.program_id(0)
    rows = o_ref.shape[1]
    classes = w1_ref.shape[0]

    @pl.when(t == 0)
    def _init():
        hist_scr[...] = jnp.zeros_like(hist_scr)
        ci = jax.lax.broadcasted_iota(jnp.int32, (classes, 128), 0)
        cj = jax.lax.broadcasted_iota(jnp.int32, (classes, 128), 1)
        vals = jnp.where(cj == 0, (ci >> 7) << 7, ci & 127)
        vals = jnp.where(cj < 2, vals, 0)
        w_scr[...] = vals.astype(jnp.bfloat16)

    @pl.when(t < tiles)
    def _stream():
        x = x_ref[0]
        hist_scr[...] += jnp.sum(x, axis=0, keepdims=True)
        d = jnp.dot(x.astype(jnp.bfloat16), w_scr[...],
                    preferred_element_type=jnp.float32)
        off = pl.multiple_of(t * rows, rows)
        idx_scr[pl.ds(off, rows), :] = jnp.sum(d, axis=1, keepdims=True)

    @pl.when(t == tiles)
    def _build_table():
        inv_n = 1.0 / jnp.sum(hist_scr[...])
        cnt_row = hist_scr[...]
        hi = jnp.floor(cnt_row * (1.0 / 64.0))
        lo = cnt_row - 64.0 * hi
        stacked = jnp.concatenate([hi, lo], axis=0)
        trow = jax.lax.broadcasted_iota(jnp.int32, (2, 128), 0)
        t_w = jnp.where(trow == 0, 64, 1).astype(jnp.bfloat16)
        cnt_full = jax.lax.dot_general(
            stacked.astype(jnp.bfloat16), t_w,
            (((0,), (0,)), ((), ())),
            preferred_element_type=jnp.float32)
        cnt = cnt_full[:, 0:1]

        a1 = a1_ref[0, 0]
        a2 = a2_ref[0, 0]

        h = w1_ref[...] + b1_ref[...]
        mean1 = jnp.sum(h * cnt, axis=0, keepdims=True) * inv_n
        d = h - mean1
        var1 = jnp.sum(d * d * cnt, axis=0, keepdims=True) * inv_n
        scale1 = jax.lax.rsqrt(var1 + EPS) * g1_ref[...]
        z = d * scale1 + be1_ref[...]
        z = jnp.where(z > 0, z, a1 * z)

        y = jnp.dot(z, w2_ref[...],
                    preferred_element_type=jnp.float32) + b2_ref[...]
        mean2 = jnp.sum(y * cnt, axis=0, keepdims=True) * inv_n
        e = y - mean2
        var2 = jnp.sum(e * e * cnt, axis=0, keepdims=True) * inv_n
        scale2 = jax.lax.rsqrt(var2 + EPS) * g2_ref[...]
        u = e * scale2 + be2_ref[...]
        table_scr[...] = jnp.where(u > 0, u, a2 * u).astype(jnp.bfloat16)

    @pl.when(t >= tiles)
    def _gather():
        off = pl.multiple_of((t - tiles) * rows, rows)
        iv = idx_scr[pl.ds(off, rows), :].astype(jnp.int32)
        lane = jax.lax.broadcasted_iota(jnp.int32, (rows, classes), 1)
        onehot = (lane == iv).astype(jnp.bfloat16)
        o_ref[0] = jnp.dot(onehot, table_scr[...],
                           preferred_element_type=jnp.float32)


def kernel(pos_onehot, w1, b1, g1, be1, a1, w2, b2, g2, be2, a2):
    b, l, classes = pos_onehot.shape
    out_dim = w2.shape[1]
    n = b * l

    tiles = b
    rows = l

    const = lambda shape: pl.BlockSpec(shape, lambda i, s=len(shape): (0,) * s)
    smem = pl.BlockSpec(memory_space=pltpu.MemorySpace.SMEM)

    import functools
    body = functools.partial(_fused_kernel, tiles)

    out = pl.pallas_call(
        body,
        out_shape=jax.ShapeDtypeStruct((b, l, out_dim), jnp.float32),
        grid=(2 * tiles,),
        in_specs=[
            pl.BlockSpec((1, l, classes),
                         lambda t: (jnp.minimum(t, tiles - 1), 0, 0)),
            const(w1.shape),
            const(b1.shape), const(g1.shape), const(be1.shape),
            const(w2.shape), const(b2.shape), const(g2.shape),
            const(be2.shape), smem, smem,
        ],
        out_specs=pl.BlockSpec((1, l, out_dim),
                               lambda t: (jnp.maximum(t - tiles, 0), 0, 0)),
        scratch_shapes=[pltpu.VMEM((n, 1), jnp.float32),
                        pltpu.VMEM((1, classes), jnp.float32),
                        pltpu.VMEM((classes, out_dim), jnp.bfloat16),
                        pltpu.VMEM((classes, 128), jnp.bfloat16)],
        compiler_params=pltpu.CompilerParams(
            dimension_semantics=("arbitrary",)),
    )(pos_onehot, w1, b1, g1, be1, w2, b2, g2, be2, a1, a2)

    return out
